# baseline (speedup 1.0000x reference)
import functools

import jax
import jax.numpy as jnp
from jax import lax
from jax.experimental import pallas as pl
from jax.experimental.pallas import tpu as pltpu
from jax.experimental.pallas import tpu_sc as plsc

EMBED_DIM = 64
NBUF = 4


@functools.lru_cache(maxsize=None)
def _make_gather(batch: int, hist: int, d: int):
    info = plsc.get_sparse_core_info()
    nc, ns = info.num_cores, info.num_subcores
    nw = nc * ns
    rows_per_w = batch // nw
    assert rows_per_w * nw == batch
    assert rows_per_w % NBUF == 0 and rows_per_w // NBUF >= 2
    mesh = plsc.VectorSubcoreMesh(core_axis_name="c", subcore_axis_name="s")

    @functools.partial(
        pl.kernel,
        mesh=mesh,
        out_type=jax.ShapeDtypeStruct((batch, hist, d), jnp.float32),
        scratch_types=[
            pltpu.VMEM((NBUF, hist), jnp.int32),
            pltpu.VMEM((NBUF, hist, d), jnp.float32),
        ]
        + [pltpu.SemaphoreType.DMA] * (2 * NBUF),
        compiler_params=pltpu.CompilerParams(use_tc_tiling_on_sc=False),
    )
    def gather_kernel(table_hbm, idx_hbm, out_hbm, idx_v, rows_v, *sems):
        wid = lax.axis_index("s") * nc + lax.axis_index("c")
        base = wid * rows_per_w
        gsems = sems[:NBUF]
        wsems = sems[NBUF:]

        def stage_idx(g, b):
            pltpu.sync_copy(idx_hbm.at[base + g], idx_v.at[b])

        def start_gather(b):
            pltpu.async_copy(table_hbm.at[idx_v.at[b]], rows_v.at[b], gsems[b])

        def wait_gather(b):
            pltpu.make_async_copy(table_hbm.at[idx_v.at[b]],
                                  rows_v.at[b], gsems[b]).wait()

        def start_write(g, b):
            pltpu.async_copy(rows_v.at[b], out_hbm.at[base + g], wsems[b])

        def wait_write(g, b):
            pltpu.make_async_copy(rows_v.at[b],
                                  out_hbm.at[base + g], wsems[b]).wait()

        for b in range(NBUF):
            stage_idx(b, b)
            start_gather(b)

        def outer(t, carry):
            for b in range(NBUF):
                g = NBUF * t + b
                wait_gather(b)
                start_write(g, b)
                stage_idx(g + NBUF, b)
                wait_write(g, b)
                start_gather(b)
            return carry

        lax.fori_loop(0, rows_per_w // NBUF - 1, outer, 0)

        for b in range(NBUF):
            g = rows_per_w - NBUF + b
            wait_gather(b)
            start_write(g, b)
        for b in range(NBUF):
            g = rows_per_w - NBUF + b
            wait_write(g, b)

    return gather_kernel


def kernel(input_ids, table):
    b, h = input_ids.shape
    ids = input_ids.astype(jnp.int32)
    return _make_gather(b, h, EMBED_DIM)(table, ids)
